# baseline (speedup 1.0000x reference)
import functools

import jax
import jax.numpy as jnp
from jax import lax
from jax.experimental import pallas as pl
from jax.experimental.pallas import tpu as pltpu, tpu_sc as plsc

IN_DIM = 1_000_000
OUT_DIM = 32
BATCH = 4096
HIST = 50

_NC = 2
_NS = 16
_NW = _NC * _NS

_B = BATCH * HIST
_CH = 128
_CPW = _B // (_NW * _CH)
_CPW_PAD = 56


def _make_kernel():
  mesh = plsc.VectorSubcoreMesh(core_axis_name="c", subcore_axis_name="s")

  @functools.partial(
      pl.kernel,
      out_type=jax.ShapeDtypeStruct((_B, OUT_DIM), jnp.float32),
      mesh=mesh,
      compiler_params=pltpu.CompilerParams(use_tc_tiling_on_sc=False),
      scratch_types=[
          pltpu.VMEM((_CPW_PAD, _CH), jnp.int32),
          pltpu.VMEM((2, _CH, OUT_DIM), jnp.float32),
          pltpu.SemaphoreType.DMA,
          pltpu.SemaphoreType.DMA,
      ],
  )
  def k(idx_hbm, table_hbm, out_hbm, idx_v, rows_v, sem0, sem1):
    wid = lax.axis_index("s") * _NC + lax.axis_index("c")
    base_chunk = wid * _CPW
    pltpu.sync_copy(idx_hbm.at[pl.ds(wid * _CPW_PAD, _CPW_PAD)], idx_v)
    idx_rows = idx_v

    sems = (sem0, sem1)
    pltpu.async_copy(table_hbm.at[idx_rows.at[0]], rows_v.at[0], sems[0])

    @pl.loop(0, _CPW, step=2)
    def _(j):
      for b in range(2):
        jj = j + b
        nxt = jj + 1

        @pl.when(nxt < _CPW)
        def _():
          pltpu.async_copy(
              table_hbm.at[idx_rows.at[nxt]], rows_v.at[(b + 1) % 2],
              sems[(b + 1) % 2])

        pltpu.make_async_copy(
            table_hbm.at[idx_rows.at[jj]], rows_v.at[b], sems[b]).wait()
        pltpu.sync_copy(
            rows_v.at[b], out_hbm.at[pl.ds((base_chunk + jj) * _CH, _CH)])

  return k


_gather = _make_kernel()


@jax.jit
def kernel(net, weights):
  idx = net.reshape(_NW, _CPW, _CH).astype(jnp.int32)
  idx = jnp.pad(idx, ((0, 0), (0, _CPW_PAD - _CPW), (0, 0)))
  idx = idx.reshape(_NW * _CPW_PAD, _CH)
  out = _gather(idx, weights)
  return out.reshape(BATCH, HIST, OUT_DIM)


# speedup vs baseline: 1.0145x; 1.0145x over previous
import functools

import jax
import jax.numpy as jnp
from jax import lax
from jax.experimental import pallas as pl
from jax.experimental.pallas import tpu as pltpu, tpu_sc as plsc

IN_DIM = 1_000_000
OUT_DIM = 32
BATCH = 4096
HIST = 50

_NC = 2
_NS = 16
_NW = _NC * _NS

_B = BATCH * HIST
_CH = 128
_CPW = _B // (_NW * _CH)
_CPW_PAD = 56
_CPG = 10
_G = _CPW // _CPG


def _make_kernel():
  mesh = plsc.VectorSubcoreMesh(core_axis_name="c", subcore_axis_name="s")

  @functools.partial(
      pl.kernel,
      out_type=jax.ShapeDtypeStruct((_B, OUT_DIM), jnp.float32),
      mesh=mesh,
      compiler_params=pltpu.CompilerParams(use_tc_tiling_on_sc=False),
      scratch_types=[
          pltpu.VMEM((_CPW_PAD, _CH), jnp.int32),
          pltpu.VMEM((2, _CPG * _CH, OUT_DIM), jnp.float32),
          pltpu.SemaphoreType.DMA,
          pltpu.SemaphoreType.DMA,
      ],
  )
  def k(idx_hbm, table_hbm, out_hbm, idx_v, rows_v, sem0, sem1):
    wid = lax.axis_index("s") * _NC + lax.axis_index("c")
    base_row = wid * _CPW * _CH
    pltpu.sync_copy(idx_hbm.at[pl.ds(wid * _CPW_PAD, _CPW_PAD)], idx_v)

    sems = (sem0, sem1)

    def fire(g, buf):
      @pl.loop(0, _CPG)
      def _(c):
        pltpu.async_copy(
            table_hbm.at[idx_v.at[g * _CPG + c]],
            rows_v.at[buf].at[pl.ds(c * _CH, _CH)],
            sems[buf])

    def drain(g, buf):
      @pl.loop(0, _CPG)
      def _(c):
        pltpu.make_async_copy(
            table_hbm.at[idx_v.at[g * _CPG + c]],
            rows_v.at[buf].at[pl.ds(c * _CH, _CH)],
            sems[buf]).wait()

    fire(0, 0)
    for g in range(_G):
      if g + 1 < _G:
        fire(g + 1, (g + 1) % 2)
      drain(g, g % 2)
      pltpu.sync_copy(
          rows_v.at[g % 2],
          out_hbm.at[pl.ds(base_row + g * _CPG * _CH, _CPG * _CH)])

  return k


_gather = _make_kernel()


@jax.jit
def kernel(net, weights):
  idx = net.reshape(_NW, _CPW, _CH).astype(jnp.int32)
  idx = jnp.pad(idx, ((0, 0), (0, _CPW_PAD - _CPW), (0, 0)))
  idx = idx.reshape(_NW * _CPW_PAD, _CH)
  out = _gather(idx, weights)
  return out.reshape(BATCH, HIST, OUT_DIM)


# speedup vs baseline: 1.1484x; 1.1320x over previous
import functools

import jax
import jax.numpy as jnp
from jax import lax
from jax.experimental import pallas as pl
from jax.experimental.pallas import tpu as pltpu, tpu_sc as plsc

IN_DIM = 1_000_000
OUT_DIM = 32
BATCH = 4096
HIST = 50

_NC = 2
_NS = 16
_NW = _NC * _NS

_B = BATCH * HIST
_U = 128
_NU = _B // _U
_UPW = _NU // _NW
_TCB = BATCH // 128


def _make_kernel():
  mesh = plsc.VectorSubcoreMesh(core_axis_name="c", subcore_axis_name="s")

  @functools.partial(
      pl.kernel,
      out_type=jax.ShapeDtypeStruct((HIST, 4, _TCB, 8, 128), jnp.float32),
      mesh=mesh,
      compiler_params=pltpu.CompilerParams(
          use_tc_tiling_on_sc=False, needs_layout_passes=False),
      scratch_types=[
          pltpu.VMEM((_UPW * _U,), jnp.int32),
          pltpu.VMEM((2, _U), jnp.int32),
          pltpu.VMEM((2, _U), jnp.int32),
          pltpu.VMEM((2, _U, 128), jnp.float32),
          pltpu.VMEM((32, 128), jnp.float32),
          pltpu.SemaphoreType.DMA,
          pltpu.SemaphoreType.DMA,
      ],
  )
  def k(idx_hbm, table_hbm, out_hbm, idx_v, g_v, s_v, gran_v, chunk_v,
        sem0, sem1):
    wid = lax.axis_index("s") * _NC + lax.axis_index("c")
    ubase = wid * _UPW
    pltpu.sync_copy(idx_hbm.at[pl.ds(wid * _UPW * _U, _UPW * _U)], idx_v)

    sems = (sem0, sem1)

    def prep_and_fire(u_local, buf):
      @pl.loop(0, _U // 16)
      def _(t):
        r = idx_v[pl.ds(u_local * _U + t * 16, 16)]
        g_v.at[buf][pl.ds(t * 16, 16)] = lax.shift_right_logical(r, 2)
        s_v.at[buf][pl.ds(t * 16, 16)] = lax.shift_left(
            lax.bitwise_and(r, 3), 5)
      pltpu.async_copy(
          table_hbm.at[g_v.at[buf]], gran_v.at[buf], sems[buf])

    def extract_and_store(u_local, buf):
      pltpu.make_async_copy(
          table_hbm.at[g_v.at[buf]], gran_v.at[buf], sems[buf]).wait()
      gran = gran_v.at[buf]
      for t in range(_U // 16):
        rows = lax.iota(jnp.int32, 16) + t * 16
        s0 = s_v.at[buf][pl.ds(t * 16, 16)]
        for c in range(OUT_DIM):
          vals = plsc.load_gather(gran, [rows, s0 + c])
          chunk_v.at[c][pl.ds(t * 16, 16)] = vals

      u = ubase + u_local
      h = u // _TCB
      tc = lax.rem(u, _TCB)
      for tr in range(4):
        pltpu.sync_copy(
            chunk_v.at[pl.ds(tr * 8, 8)], out_hbm.at[h, tr, tc])

    prep_and_fire(0, 0)

    @pl.loop(0, _UPW, step=2)
    def _(u0):
      for b in range(2):
        u_local = u0 + b
        nxt = u_local + 1

        @pl.when(nxt < _UPW)
        def _():
          prep_and_fire(nxt, (b + 1) % 2)

        extract_and_store(u_local, b)

  return k


_gather = _make_kernel()


@jax.jit
def kernel(net, weights):
  idx = net.T.reshape(_B).astype(jnp.int32)
  table = weights.reshape(IN_DIM * OUT_DIM // 128, 128)
  out5d = _gather(idx, table)
  return out5d.transpose(2, 4, 0, 1, 3).reshape(BATCH, HIST, OUT_DIM)
